# Initial kernel scaffold; baseline (speedup 1.0000x reference)
import jax
import jax.numpy as jnp
from jax.experimental import pallas as pl


def kernel(x, emb, W_ih, W_hh, b_ih, b_hh, fc_w, fc_b):
    raise NotImplementedError("write your pallas kernel here")


# speedup vs baseline: 4.5499x; 4.5499x over previous
import functools

import jax
import jax.numpy as jnp
from jax import lax
from jax.experimental import pallas as pl
from jax.experimental.pallas import tpu as pltpu
from jax.experimental.pallas import tpu_sc as plsc

VOCAB = 100000
HIDDEN = 512
BATCH = 64
SEQ = 200


def _sc_gather(emb, idx_flat):
    info = plsc.get_sparse_core_info()
    nw = info.num_cores * info.num_subcores
    n = idx_flat.shape[0]
    per_w = n // nw
    ch = 80
    nch = per_w // ch
    mesh = plsc.VectorSubcoreMesh(core_axis_name="c", subcore_axis_name="s")

    @functools.partial(
        pl.kernel,
        mesh=mesh,
        out_type=jax.ShapeDtypeStruct((n, HIDDEN), jnp.float32),
        scratch_types=[
            pltpu.VMEM((ch,), jnp.int32),
            pltpu.VMEM((ch, HIDDEN), jnp.float32),
            pltpu.SemaphoreType.DMA,
        ],
    )
    def gather_kernel(table_hbm, idx_hbm, out_hbm, idx_v, rows_v, sem):
        wid = lax.axis_index("s") * info.num_cores + lax.axis_index("c")
        base = wid * per_w
        for c in range(nch):
            off = base + c * ch
            pltpu.sync_copy(idx_hbm.at[pl.ds(off, ch)], idx_v)
            pltpu.async_copy(table_hbm.at[idx_v], rows_v, sem).wait()
            pltpu.sync_copy(rows_v, out_hbm.at[pl.ds(off, ch)])

    return gather_kernel(emb, idx_flat)


def _rnn_step(e_ref, wi1t, wh1t, wi2t, wh2t, b1, b2, fct, fcb,
              out_ref, h1_ref, h2_ref):
    t = pl.program_id(0)

    @pl.when(t == 0)
    def _init():
        h1_ref[...] = jnp.zeros_like(h1_ref)
        h2_ref[...] = jnp.zeros_like(h2_ref)

    e = e_ref[0]
    a1 = jnp.dot(e, wi1t[...], preferred_element_type=jnp.float32) + b1[...]
    h1 = jnp.tanh(a1 + jnp.dot(h1_ref[...], wh1t[...],
                               preferred_element_type=jnp.float32))
    a2 = jnp.dot(h1, wi2t[...], preferred_element_type=jnp.float32) + b2[...]
    h2 = jnp.tanh(a2 + jnp.dot(h2_ref[...], wh2t[...],
                               preferred_element_type=jnp.float32))
    h1_ref[...] = h1
    h2_ref[...] = h2

    @pl.when(t == SEQ - 1)
    def _head():
        out_ref[...] = jnp.dot(h2, fct[...],
                               preferred_element_type=jnp.float32) + fcb[...]


def _rnn_scan(e3, wi1t, wh1t, wi2t, wh2t, b1, b2, fct, fcb):
    full = lambda shape: pl.BlockSpec(shape, lambda t: (0,) * len(shape))
    return pl.pallas_call(
        _rnn_step,
        grid=(SEQ,),
        in_specs=[
            pl.BlockSpec((1, BATCH, HIDDEN), lambda t: (t, 0, 0)),
            full((HIDDEN, HIDDEN)),
            full((HIDDEN, HIDDEN)),
            full((HIDDEN, HIDDEN)),
            full((HIDDEN, HIDDEN)),
            full((1, HIDDEN)),
            full((1, HIDDEN)),
            full((HIDDEN, 2)),
            full((1, 2)),
        ],
        out_specs=full((BATCH, 2)),
        out_shape=jax.ShapeDtypeStruct((BATCH, 2), jnp.float32),
        scratch_shapes=[
            pltpu.VMEM((BATCH, HIDDEN), jnp.float32),
            pltpu.VMEM((BATCH, HIDDEN), jnp.float32),
        ],
    )(e3, wi1t, wh1t, wi2t, wh2t, b1, b2, fct, fcb)


def kernel(x, emb, W_ih, W_hh, b_ih, b_hh, fc_w, fc_b):
    x = x.astype(jnp.int32)
    idx_flat = x.T.reshape(-1)
    e = _sc_gather(emb, idx_flat)
    e3 = e.reshape(SEQ, BATCH, HIDDEN)

    wi1t = W_ih[0].T
    wh1t = W_hh[0].T
    wi2t = W_ih[1].T
    wh2t = W_hh[1].T
    b1 = (b_ih[0] + b_hh[0]).reshape(1, HIDDEN)
    b2 = (b_ih[1] + b_hh[1]).reshape(1, HIDDEN)
    fct = fc_w.T
    fcb = fc_b.reshape(1, 2)
    return _rnn_scan(e3, wi1t, wh1t, wi2t, wh2t, b1, b2, fct, fcb)
